# baseline (speedup 1.0000x reference)
import functools

import jax
import jax.numpy as jnp
from jax import lax
from jax.experimental import pallas as pl
from jax.experimental.pallas import tpu as pltpu
from jax.experimental.pallas import tpu_sc as plsc

V, E, B, K = 1000000, 64, 16384, 20
NC, NS = 2, 16
NW = NC * NS
S = B // NW
C = 16
NCH = S // C
NIR = C * K // 64
NROW = S * K // 64
EV = E // 16


def _sc_partials(center, outside, neg2d, embc2, embo2):
  mesh = plsc.VectorSubcoreMesh(core_axis_name="c", subcore_axis_name="s")

  @functools.partial(
      pl.kernel, mesh=mesh,
      out_type=jax.ShapeDtypeStruct((B, 32), jnp.float32),
      scratch_types=[
          pltpu.VMEM((S,), jnp.int32),
          pltpu.VMEM((S + 16,), jnp.int32),
          pltpu.VMEM((S,), jnp.int32),
          pltpu.VMEM((S + 16,), jnp.int32),
          pltpu.VMEM((NROW, 64), jnp.int32),
          pltpu.VMEM((S * K + 16,), jnp.int32),
          pltpu.VMEM((C, 2 * E), jnp.float32),
          pltpu.VMEM((C, 2 * E), jnp.float32),
          pltpu.VMEM((C, 2 * E), jnp.float32),
          pltpu.VMEM((C, 2 * E), jnp.float32),
          pltpu.VMEM((C * K, 2 * E), jnp.float32),
          pltpu.VMEM((C * K, 2 * E), jnp.float32),
          pltpu.VMEM((C, 32), jnp.float32),
          pltpu.VMEM((C, 32), jnp.float32),
          pltpu.SemaphoreType.DMA,
          pltpu.SemaphoreType.DMA,
          pltpu.SemaphoreType.DMA,
          pltpu.SemaphoreType.DMA,
      ])
  def k(center_hbm, outside_hbm, neg_hbm, embc_hbm, embo_hbm, out_hbm,
        cphy, coff, ophy, ooff, nphy, noff, crows0, crows1, orows0, orows1,
        nrows0, nrows1, outv0, outv1, sem0, sem1, osem0, osem1):
    crows = (crows0, crows1)
    orows = (orows0, orows1)
    nrows = (nrows0, nrows1)
    outv = (outv0, outv1)
    sems = (sem0, sem1)
    osems = (osem0, osem1)
    wid = lax.axis_index("s") * NC + lax.axis_index("c")
    base = wid * S

    pltpu.sync_copy(center_hbm.at[pl.ds(base, S)], cphy)
    pltpu.sync_copy(outside_hbm.at[pl.ds(base, S)], ophy)
    pltpu.sync_copy(neg_hbm.at[pl.ds(wid * NROW, NROW), :], nphy)

    def split_1d(ref, off_ref, t, _):
      v = ref[pl.ds(t * 16, 16)]
      off_ref[pl.ds(t * 16, 16)] = (v & 1) << 6
      ref[pl.ds(t * 16, 16)] = v >> 1
      return _

    lax.fori_loop(0, S // 16, functools.partial(split_1d, cphy, coff), 0)
    lax.fori_loop(0, S // 16, functools.partial(split_1d, ophy, ooff), 0)

    def split_neg(t, _):
      r = t // 4
      j = t % 4
      v = nphy[r, pl.ds(j * 16, 16)]
      noff[pl.ds(t * 16, 16)] = (v & 1) << 6
      nphy[r, pl.ds(j * 16, 16)] = v >> 1
      return _

    lax.fori_loop(0, NROW * 4, split_neg, 0)

    def issue(g, p):
      pltpu.async_copy(embc_hbm.at[cphy.at[pl.ds(g * C, C)]], crows[p],
                       sems[p])
      pltpu.async_copy(embo_hbm.at[ophy.at[pl.ds(g * C, C)]], orows[p],
                       sems[p])
      for j in range(NIR):
        pltpu.async_copy(embo_hbm.at[nphy.at[g * NIR + j]],
                         nrows[p].at[pl.ds(j * 64, 64)], sems[p])

    def wait(p):
      pltpu.make_async_copy(embc_hbm.at[pl.ds(0, C)], crows[p],
                            sems[p]).wait()
      pltpu.make_async_copy(embc_hbm.at[pl.ds(0, C)], orows[p],
                            sems[p]).wait()
      for j in range(NIR):
        pltpu.make_async_copy(embc_hbm.at[pl.ds(0, 64)],
                              nrows[p].at[pl.ds(j * 64, 64)],
                              sems[p]).wait()

    def compute(g, p):
      cr, orr, nr, ov = crows[p], orows[p], nrows[p], outv[p]

      def body2(lb, carry):
        bg = g * C + lb
        co = coff[pl.ds(bg, 16)][0]
        oo = ooff[pl.ds(bg, 16)][0]
        nov0 = noff[pl.ds(bg * K, 16)]
        nov1 = noff[pl.ds(bg * K + 16, 16)]
        nos = [nov0[kk] for kk in range(16)] + [nov1[kk] for kk in range(4)]
        cs = [cr[lb, pl.ds(co + 16 * j, 16)] for j in range(EV)]
        acc_o = cs[0] * orr[lb, pl.ds(oo, 16)]
        for j in range(1, EV):
          acc_o = acc_o + cs[j] * orr[lb, pl.ds(oo + 16 * j, 16)]
        acc_n = None
        for j in range(EV):
          s = None
          for kk in range(K):
            r = nr[lb * K + kk, pl.ds(nos[kk] + 16 * j, 16)]
            s = r if s is None else s + r
          t = s * cs[j]
          acc_n = t if acc_n is None else acc_n + t
        ov[lb, pl.ds(0, 16)] = acc_o
        ov[lb, pl.ds(16, 16)] = acc_n
        return carry

      lax.fori_loop(0, C, body2, 0)

    def flush(g, p):
      pltpu.async_copy(outv[p], out_hbm.at[pl.ds(base + g * C, C), :],
                       osems[p])

    def owait(p):
      pltpu.make_async_copy(out_hbm.at[pl.ds(0, C), :], outv[p],
                            osems[p]).wait()

    issue(0, 0)

    def outer(gp, carry):
      for lane in range(2):
        g = gp * 2 + lane

        @pl.when(g + 1 < NCH)
        def _():
          issue(g + 1, (lane + 1) % 2)

        wait(lane)

        @pl.when(g >= 2)
        def _():
          owait(lane)

        compute(g, lane)
        flush(g, lane)
      return carry

    lax.fori_loop(0, NCH // 2, outer, 0)
    owait(0)
    owait(1)

  return k(center, outside, neg2d, embc2, embo2)


def _logsig(x):
  return jnp.minimum(x, 0.0) - jnp.log1p(jnp.exp(-jnp.abs(x)))


def _finish_body(p_ref, o_ref):
  x = p_ref[...]
  uovc = jnp.sum(x[:, 0:16], axis=1)
  nd = jnp.sum(x[:, 16:32], axis=1)
  loss = _logsig(uovc) + _logsig(-nd)
  o_ref[...] = jnp.broadcast_to(-jnp.mean(loss), (1, 1))


def kernel(center, outside, negative, emb_center, emb_outside):
  c = center.reshape(B).astype(jnp.int32)
  o = outside.reshape(B).astype(jnp.int32)
  n = negative.reshape(B * K // 64, 64).astype(jnp.int32)
  embc2 = emb_center.reshape(V // 2, 2 * E)
  embo2 = emb_outside.reshape(V // 2, 2 * E)
  parts = _sc_partials(c, o, n, embc2, embo2)
  out = pl.pallas_call(
      _finish_body,
      out_shape=jax.ShapeDtypeStruct((1, 1), jnp.float32))(parts)
  return out[0, 0]
